# baseline (speedup 1.0000x reference)
import functools

import jax
import jax.numpy as jnp
from jax import lax
from jax.experimental import pallas as pl
from jax.experimental.pallas import tpu as pltpu
import jax.experimental.pallas.tpu_sc as plsc

D = 1024
H = 768
T = 4096
NE = 23
K = 3

TK = T * K
BLK = 128
NB = 120
NPAD = NB * BLK
NW = 32
LANES = 128
TBLK = 512
RB = 256

_NEG = -1e30


def _gelu(v):
    return 0.5 * v * (1.0 + lax.erf(v * (2.0 ** -0.5)))


def _gate_body(x_ref, wg_ref, bg_ref, bias_ref,
               i0, i1, i2, w0, w1, w2, pcol_ref, cnt_ref):
    x = x_ref[...]
    logits = jnp.dot(x, wg_ref[...], preferred_element_type=jnp.float32)
    logits = logits + bg_ref[...]
    lane = lax.broadcasted_iota(jnp.int32, (TBLK, LANES), 1)
    valid = lane < NE
    g = jnp.where(valid, jax.nn.sigmoid(logits), 0.0)
    scores = jnp.where(valid, logits + bias_ref[...], _NEG)

    idxs = []
    vals = []
    work = scores
    cnt = jnp.zeros((1, LANES), jnp.float32)
    for _ in range(K):
        m = jnp.max(work, axis=1, keepdims=True)
        hit = work == m
        idx = jnp.min(jnp.where(hit, lane, LANES), axis=1, keepdims=True)
        sel = lane == idx
        gval = jnp.sum(jnp.where(sel, g, 0.0), axis=1, keepdims=True)
        cnt = cnt + jnp.sum(sel.astype(jnp.float32), axis=0, keepdims=True)
        idxs.append(idx)
        vals.append(gval)
        work = jnp.where(sel, _NEG, work)

    wsum = vals[0] + vals[1] + vals[2]
    i0[...], i1[...], i2[...] = idxs[0], idxs[1], idxs[2]
    w0[...] = vals[0] / wsum
    w1[...] = vals[1] / wsum
    w2[...] = vals[2] / wsum

    gn = g / jnp.sum(g, axis=1, keepdims=True)
    part = jnp.sum(gn, axis=0, keepdims=True)

    @pl.when(pl.program_id(0) == 0)
    def _():
        pcol_ref[...] = jnp.zeros_like(pcol_ref)
        cnt_ref[...] = jnp.zeros_like(cnt_ref)

    pcol_ref[...] += part
    cnt_ref[...] += cnt


def _gate_topk(x, wg_p, bg_p, bias_p):
    grid = T // TBLK
    col = jax.ShapeDtypeStruct((T, 1), jnp.float32)
    coli = jax.ShapeDtypeStruct((T, 1), jnp.int32)
    out_shape = (coli, coli, coli, col, col, col,
                 jax.ShapeDtypeStruct((1, LANES), jnp.float32),
                 jax.ShapeDtypeStruct((1, LANES), jnp.float32))
    colspec = pl.BlockSpec((TBLK, 1), lambda i: (i, 0))
    return pl.pallas_call(
        _gate_body,
        grid=(grid,),
        in_specs=[
            pl.BlockSpec((TBLK, D), lambda i: (i, 0)),
            pl.BlockSpec((D, LANES), lambda i: (0, 0)),
            pl.BlockSpec((1, LANES), lambda i: (0, 0)),
            pl.BlockSpec((1, LANES), lambda i: (0, 0)),
        ],
        out_specs=(colspec, colspec, colspec, colspec, colspec, colspec,
                   pl.BlockSpec((1, LANES), lambda i: (0, 0)),
                   pl.BlockSpec((1, LANES), lambda i: (0, 0))),
        out_shape=out_shape,
        compiler_params=pltpu.CompilerParams(
            dimension_semantics=("arbitrary",)),
    )(x, wg_p, bg_p, bias_p)


def _hi(a, b):
    return jnp.dot(a, b, preferred_element_type=jnp.float32,
                   precision=lax.Precision.HIGHEST)


def _dispatch_body(ti_ref, pcol_ref, cnt_ref, dest_ref, be_ref, nreal_ref,
                   aux_ref):
    n_blocks = TK // RB
    row_i = lax.broadcasted_iota(jnp.int32, (LANES, LANES), 0)
    col_i = lax.broadcasted_iota(jnp.int32, (LANES, LANES), 1)
    row_b = lax.broadcasted_iota(jnp.int32, (RB, RB), 0)
    col_b = lax.broadcasted_iota(jnp.int32, (RB, RB), 1)
    l_strict = (col_b < row_b).astype(jnp.float32)
    u_strict = (row_i < col_i).astype(jnp.float32)
    ident = (row_i == col_i).astype(jnp.float32)
    ones_col = jnp.ones((LANES, 1), jnp.float32)
    lane_i = lax.broadcasted_iota(jnp.int32, (1, LANES), 1)
    lane_row = lane_i.astype(jnp.float32)
    iota_col = lax.broadcasted_iota(jnp.int32, (LANES, 1), 0) \
        .astype(jnp.float32)

    def t_col(row):
        return lax.dot_general(ident, row, (((1,), (1,)), ((), ())),
                               precision=lax.Precision.HIGHEST,
                               preferred_element_type=jnp.float32)

    def onehot_at(b):
        return (ti_ref[b] == lane_i).astype(jnp.float32)

    counts = cnt_ref[...]
    ci = counts.astype(jnp.int32)
    padded = ((ci + (BLK - 1)) // BLK) * BLK
    padded = jnp.where(lane_i >= NE, 0, padded)
    ptot = jnp.sum(padded, axis=1, keepdims=True)
    padded = padded + jnp.where(lane_i == NE, NPAD - ptot, 0)
    padded_f = padded.astype(jnp.float32)
    offsets = _hi(padded_f, u_strict)
    offsets_col = t_col(offsets)

    def p2(b, carry):
        oh = onehot_at(b)
        cum = jnp.dot(l_strict, oh,
                      preferred_element_type=jnp.float32)
        rank = jnp.sum(oh * (cum + carry), axis=1, keepdims=True)
        dest = rank + jnp.dot(oh, offsets_col,
                              preferred_element_type=jnp.float32)
        dest_ref[b] = dest.astype(jnp.int32)
        return carry + jnp.sum(oh, axis=0, keepdims=True)

    lax.fori_loop(0, n_blocks, p2, jnp.zeros((1, LANES), jnp.float32))

    bs = offsets / float(BLK)
    mje = jnp.logical_and(iota_col >= bs, lane_row < NE).astype(jnp.float32)
    be = _hi(mje, ones_col) - 1.0
    be_ref[...] = jnp.clip(be, 0.0, NE - 1.0).astype(jnp.int32)
    nreal_ref[...] = (ptot // BLK).astype(jnp.int32)

    scale = NE / (float(K) * T * T)
    aux_ref[...] = jnp.sum(pcol_ref[...] * counts,
                           axis=1, keepdims=True) * scale


def _dispatch(ti3d, pcol, cnt):
    return pl.pallas_call(
        _dispatch_body,
        out_shape=(
            jax.ShapeDtypeStruct((TK // RB, RB, 1), jnp.int32),
            jax.ShapeDtypeStruct((LANES, 1), jnp.int32),
            jax.ShapeDtypeStruct((1, 1), jnp.int32),
            jax.ShapeDtypeStruct((1, 1), jnp.float32),
        ),
    )(ti3d, pcol, cnt)


_STRIPE = NPAD // 16
_NCH = 12


def _scatter_body(x_hbm, dest3_hbm, destw_hbm, tww_hbm,
                  xg_hbm, wpart_hbm,
                  didx_v, didxw_v, tww_v, rows_v, rows2_v, wz_v,
                  sem, sem2, semw, spw):
    cid = lax.axis_index("c")
    sid = lax.axis_index("s")
    wid = sid * 2 + cid
    pltpu.sync_copy(dest3_hbm.at[wid], didx_v)
    pltpu.sync_copy(destw_hbm.at[wid], didxw_v)
    pltpu.sync_copy(tww_hbm.at[wid], tww_v)
    for i in range(_STRIPE // 16):
        wz_v[pl.ds(i * 16, 16)] = jnp.zeros((16,), jnp.float32)
    pltpu.sync_copy(wz_v, spw.at[pl.ds(sid * _STRIPE, _STRIPE)])
    plsc.subcore_barrier()
    wcps = [pltpu.async_copy(tww_v.at[j], spw.at[didxw_v.at[j]], semw,
                             add=True)
            for j in range(K)]
    rows = (rows_v, rows2_v)
    sems = (sem, sem2)
    pltpu.sync_copy(x_hbm.at[pl.ds(wid * 128, 32)], rows_v)
    pending = {}
    for c in range(4):
        cur = rows[c % 2]
        pending[c] = [
            pltpu.async_copy(cur, xg_hbm.at[didx_v.at[k * 4 + c]],
                             sems[c % 2])
            for k in range(K)]
        if c + 1 < 4:
            if c - 1 in pending:
                for cp in pending.pop(c - 1):
                    cp.wait()
            pltpu.sync_copy(x_hbm.at[pl.ds(wid * 128 + (c + 1) * 32, 32)],
                            rows[(c + 1) % 2])
    for c in sorted(pending):
        for cp in pending[c]:
            cp.wait()
    for cp in wcps:
        cp.wait()
    plsc.subcore_barrier()
    pltpu.sync_copy(spw.at[pl.ds(sid * _STRIPE, _STRIPE)], wz_v)
    off = pl.multiple_of(cid * NPAD + sid * _STRIPE, 8)
    pltpu.sync_copy(wz_v, wpart_hbm.at[pl.ds(off, _STRIPE)])


def _scatter_rows(x, dest3, destw, tww):
    return pl.kernel(
        _scatter_body,
        out_type=(
            jax.ShapeDtypeStruct((NPAD, D), jnp.float32),
            jax.ShapeDtypeStruct((2 * NPAD,), jnp.float32),
        ),
        mesh=plsc.VectorSubcoreMesh(core_axis_name="c", subcore_axis_name="s"),
        scratch_types=[
            pltpu.VMEM((_NCH, 32), jnp.int32),
            pltpu.VMEM((K, LANES), jnp.int32),
            pltpu.VMEM((K, LANES), jnp.float32),
            pltpu.VMEM((32, D), jnp.float32),
            pltpu.VMEM((32, D), jnp.float32),
            pltpu.VMEM((_STRIPE,), jnp.float32),
            pltpu.SemaphoreType.DMA,
            pltpu.SemaphoreType.DMA,
            pltpu.SemaphoreType.DMA,
            pltpu.VMEM_SHARED((NPAD,), jnp.float32),
        ],
    )(x, dest3, destw, tww)


def _expert_body(be_ref, nb_ref, xg_ref, w1a_ref, b1a_ref, w2a_ref, b2a_ref,
                 w1b_ref, b1b_ref, w2b_ref, b2b_ref, wv_ref, out_ref):
    def half(lo, w1, b1, w2, b2):
        x = xg_ref[pl.ds(lo, BLK), :]
        h = jnp.dot(x, w1[0], preferred_element_type=jnp.float32)
        h = _gelu(h + b1[0])
        o = jnp.dot(h, w2[0], preferred_element_type=jnp.float32)
        o = o + b2[0]
        out_ref[pl.ds(lo, BLK), :] = o * (wv_ref[0, 0, pl.ds(lo, BLK)]
                                          + wv_ref[1, 0, pl.ds(lo, BLK)])

    @pl.when(2 * pl.program_id(0) < nb_ref[0])
    def _():
        half(0, w1a_ref, b1a_ref, w2a_ref, b2a_ref)

    @pl.when(2 * pl.program_id(0) + 1 < nb_ref[0])
    def _():
        half(BLK, w1b_ref, b1b_ref, w2b_ref, b2b_ref)


def _expert_mlp(be, nb, xg, wu1, bu1, wu2, bu2, wpart4d):
    wspecs = [
        pl.BlockSpec((1, D, H), lambda i, be, nb: (be[2 * i], 0, 0)),
        pl.BlockSpec((1, 1, H), lambda i, be, nb: (be[2 * i], 0, 0)),
        pl.BlockSpec((1, H, D), lambda i, be, nb: (be[2 * i], 0, 0)),
        pl.BlockSpec((1, 1, D), lambda i, be, nb: (be[2 * i], 0, 0)),
        pl.BlockSpec((1, D, H), lambda i, be, nb: (be[2 * i + 1], 0, 0)),
        pl.BlockSpec((1, 1, H), lambda i, be, nb: (be[2 * i + 1], 0, 0)),
        pl.BlockSpec((1, H, D), lambda i, be, nb: (be[2 * i + 1], 0, 0)),
        pl.BlockSpec((1, 1, D), lambda i, be, nb: (be[2 * i + 1], 0, 0)),
    ]
    grid_spec = pltpu.PrefetchScalarGridSpec(
        num_scalar_prefetch=2,
        grid=(NB // 2,),
        in_specs=[
            pl.BlockSpec((2 * BLK, D), lambda i, be, nb: (i, 0)),
            *wspecs,
            pl.BlockSpec((2, 1, 2 * BLK, 1), lambda i, be, nb: (0, i, 0, 0)),
        ],
        out_specs=pl.BlockSpec((2 * BLK, D), lambda i, be, nb: (i, 0)),
    )
    b1r = bu1.reshape(NE, 1, H)
    b2r = bu2.reshape(NE, 1, D)
    return pl.pallas_call(
        _expert_body,
        grid_spec=grid_spec,
        out_shape=jax.ShapeDtypeStruct((NPAD, D), jnp.float32),
        compiler_params=pltpu.CompilerParams(
            dimension_semantics=("arbitrary",)),
    )(be, nb, xg, wu1, b1r, wu2, b2r, wu1, b1r, wu2, b2r, wpart4d)


def _shared_body(x_ref, w1_ref, b1_ref, w2_ref, b2_ref, out_ref):
    h = jnp.dot(x_ref[...], w1_ref[...], preferred_element_type=jnp.float32)
    h = _gelu(h + b1_ref[...])
    o = jnp.dot(h, w2_ref[...], preferred_element_type=jnp.float32)
    out_ref[...] = o + b2_ref[...]


def _shared_mlp(x, ws1, bs1, ws2, bs2):
    return pl.pallas_call(
        _shared_body,
        grid=(T // TBLK,),
        in_specs=[
            pl.BlockSpec((TBLK, D), lambda i: (i, 0)),
            pl.BlockSpec((D, H), lambda i: (0, 0)),
            pl.BlockSpec((1, H), lambda i: (0, 0)),
            pl.BlockSpec((H, D), lambda i: (0, 0)),
            pl.BlockSpec((1, D), lambda i: (0, 0)),
        ],
        out_specs=pl.BlockSpec((TBLK, D), lambda i: (i, 0)),
        out_shape=jax.ShapeDtypeStruct((T, D), jnp.float32),
    )(x, ws1, bs1.reshape(1, H), ws2, bs2.reshape(1, D))


_CTOK = 16


def _combine_body(dest_hbm, oute_hbm, shared_hbm, res_hbm,
                  didx_v, g0, g1, acc_v, sem0, sem1):
    wid = lax.axis_index("s") * 2 + lax.axis_index("c")
    per_tok = T // NW
    n_chunks = per_tok // _CTOK
    pltpu.sync_copy(dest_hbm.at[pl.ds(wid * per_tok * K, per_tok * K)],
                    didx_v)
    bufs = (g0, g1)
    sems = (sem0, sem1)

    def start(c):
        return pltpu.async_copy(
            oute_hbm.at[didx_v.at[pl.ds(c * _CTOK * K, _CTOK * K)]],
            bufs[c % 2], sems[c % 2])

    cps = {0: start(0)}
    for c in range(n_chunks):
        if c + 1 < n_chunks:
            cps[c + 1] = start(c + 1)
        tbase = wid * per_tok + c * _CTOK
        pltpu.sync_copy(shared_hbm.at[pl.ds(tbase, _CTOK)], acc_v)
        cps[c].wait()
        gath_v = bufs[c % 2]

        @plsc.parallel_loop(0, _CTOK, unroll=2)
        def add_tok(u):
            for j in range(D // 16):
                s = pl.ds(j * 16, 16)
                acc_v[u, s] = (acc_v[u, s] + gath_v[u, s]
                               + gath_v[_CTOK + u, s]
                               + gath_v[2 * _CTOK + u, s])

        pltpu.sync_copy(acc_v, res_hbm.at[pl.ds(tbase, _CTOK)])


def _combine(dest_flat, oute, shared):
    return pl.kernel(
        _combine_body,
        out_type=jax.ShapeDtypeStruct((T, D), jnp.float32),
        mesh=plsc.VectorSubcoreMesh(core_axis_name="c", subcore_axis_name="s"),
        scratch_types=[
            pltpu.VMEM((T // NW * K,), jnp.int32),
            pltpu.VMEM((_CTOK * K, D), jnp.float32),
            pltpu.VMEM((_CTOK * K, D), jnp.float32),
            pltpu.VMEM((_CTOK, D), jnp.float32),
            pltpu.SemaphoreType.DMA,
            pltpu.SemaphoreType.DMA,
        ],
    )(dest_flat, oute, shared)


@jax.jit
def kernel(x, Wg, bg, Wu1, bu1, Wu2, bu2, Ws1, bs1, Ws2, bs2, bias):
    o_shape = x.shape
    xf = x.reshape(-1, o_shape[-1])

    wg_p = jnp.zeros((D, LANES), jnp.float32).at[:, :NE].set(Wg)
    bg_p = jnp.zeros((1, LANES), jnp.float32).at[0, :NE].set(bg)
    bias_p = jnp.full((1, LANES), _NEG, jnp.float32).at[0, :NE].set(bias)

    i0, i1, i2, w0, w1, w2, pcol, cnt = _gate_topk(xf, wg_p, bg_p, bias_p)

    ti = jnp.concatenate([i0, i1, i2], axis=1).reshape(-1)
    tw = jnp.concatenate([w0, w1, w2], axis=1).reshape(-1)

    dest, be, nreal, aux = _dispatch(ti.reshape(TK // RB, RB, 1),
                                     pcol, cnt)
    dest_flat = dest.reshape(-1)

    dest3 = dest_flat.reshape(T, K).T.reshape(K, NW, 4, 32) \
        .transpose(1, 0, 2, 3).reshape(NW, _NCH, 32)
    destw = dest_flat.reshape(NW, K, LANES)
    tww = tw.reshape(NW, K, LANES)

    shared = _shared_mlp(xf, Ws1, bs1, Ws2, bs2)
    xg, wpart = _scatter_rows(xf, dest3, destw, tww)

    n_ch = (T // NW) // _CTOK
    dest_g = dest_flat.reshape(NW, n_ch, _CTOK, K).transpose(0, 1, 3, 2) \
        .reshape(-1)

    oute = _expert_mlp(be.reshape(-1)[:NB], nreal.reshape(-1), xg,
                       Wu1, bu1, Wu2, bu2,
                       wpart.reshape(2, NB // 2, 2 * BLK, 1))
    result = _combine(dest_g, oute, shared)

    return result.reshape(o_shape), aux[0, 0]


# speedup vs baseline: 1.0180x; 1.0180x over previous
import functools

import jax
import jax.numpy as jnp
from jax import lax
from jax.experimental import pallas as pl
from jax.experimental.pallas import tpu as pltpu
import jax.experimental.pallas.tpu_sc as plsc

D = 1024
H = 768
T = 4096
NE = 23
K = 3

TK = T * K
BLK = 128
NB = 120
NPAD = NB * BLK
NW = 32
LANES = 128
TBLK = 512
RB = 256

_NEG = -1e30


def _gelu(v):
    return 0.5 * v * (1.0 + lax.erf(v * (2.0 ** -0.5)))


def _gate_body(x_ref, wg_ref, bg_ref, bias_ref,
               i0, i1, i2, w0, w1, w2, pcol_ref, cnt_ref):
    x = x_ref[...]
    logits = jnp.dot(x, wg_ref[...], preferred_element_type=jnp.float32)
    logits = logits + bg_ref[...]
    lane = lax.broadcasted_iota(jnp.int32, (TBLK, LANES), 1)
    valid = lane < NE
    g = jnp.where(valid, jax.nn.sigmoid(logits), 0.0)
    scores = jnp.where(valid, logits + bias_ref[...], _NEG)

    idxs = []
    vals = []
    work = scores
    cnt = jnp.zeros((1, LANES), jnp.float32)
    for _ in range(K):
        m = jnp.max(work, axis=1, keepdims=True)
        hit = work == m
        idx = jnp.min(jnp.where(hit, lane, LANES), axis=1, keepdims=True)
        sel = lane == idx
        gval = jnp.sum(jnp.where(sel, g, 0.0), axis=1, keepdims=True)
        cnt = cnt + jnp.sum(sel.astype(jnp.float32), axis=0, keepdims=True)
        idxs.append(idx)
        vals.append(gval)
        work = jnp.where(sel, _NEG, work)

    wsum = vals[0] + vals[1] + vals[2]
    i0[...], i1[...], i2[...] = idxs[0], idxs[1], idxs[2]
    w0[...] = vals[0] / wsum
    w1[...] = vals[1] / wsum
    w2[...] = vals[2] / wsum

    gn = g / jnp.sum(g, axis=1, keepdims=True)
    part = jnp.sum(gn, axis=0, keepdims=True)

    @pl.when(pl.program_id(0) == 0)
    def _():
        pcol_ref[...] = jnp.zeros_like(pcol_ref)
        cnt_ref[...] = jnp.zeros_like(cnt_ref)

    pcol_ref[...] += part
    cnt_ref[...] += cnt


def _gate_topk(x, wg_p, bg_p, bias_p):
    grid = T // TBLK
    col = jax.ShapeDtypeStruct((T, 1), jnp.float32)
    coli = jax.ShapeDtypeStruct((T, 1), jnp.int32)
    out_shape = (coli, coli, coli, col, col, col,
                 jax.ShapeDtypeStruct((1, LANES), jnp.float32),
                 jax.ShapeDtypeStruct((1, LANES), jnp.float32))
    colspec = pl.BlockSpec((TBLK, 1), lambda i: (i, 0))
    return pl.pallas_call(
        _gate_body,
        grid=(grid,),
        in_specs=[
            pl.BlockSpec((TBLK, D), lambda i: (i, 0)),
            pl.BlockSpec((D, LANES), lambda i: (0, 0)),
            pl.BlockSpec((1, LANES), lambda i: (0, 0)),
            pl.BlockSpec((1, LANES), lambda i: (0, 0)),
        ],
        out_specs=(colspec, colspec, colspec, colspec, colspec, colspec,
                   pl.BlockSpec((1, LANES), lambda i: (0, 0)),
                   pl.BlockSpec((1, LANES), lambda i: (0, 0))),
        out_shape=out_shape,
        compiler_params=pltpu.CompilerParams(
            dimension_semantics=("arbitrary",)),
    )(x, wg_p, bg_p, bias_p)


def _hi(a, b):
    return jnp.dot(a, b, preferred_element_type=jnp.float32,
                   precision=lax.Precision.HIGHEST)


def _dispatch_body(ti_ref, pcol_ref, cnt_ref, dest_ref, be_ref, nreal_ref,
                   aux_ref):
    n_blocks = TK // RB
    row_i = lax.broadcasted_iota(jnp.int32, (LANES, LANES), 0)
    col_i = lax.broadcasted_iota(jnp.int32, (LANES, LANES), 1)
    row_b = lax.broadcasted_iota(jnp.int32, (RB, RB), 0)
    col_b = lax.broadcasted_iota(jnp.int32, (RB, RB), 1)
    l_strict = (col_b < row_b).astype(jnp.float32)
    u_strict = (row_i < col_i).astype(jnp.float32)
    ident = (row_i == col_i).astype(jnp.float32)
    ones_col = jnp.ones((LANES, 1), jnp.float32)
    lane_i = lax.broadcasted_iota(jnp.int32, (1, LANES), 1)
    lane_row = lane_i.astype(jnp.float32)
    iota_col = lax.broadcasted_iota(jnp.int32, (LANES, 1), 0) \
        .astype(jnp.float32)

    def t_col(row):
        return lax.dot_general(ident, row, (((1,), (1,)), ((), ())),
                               precision=lax.Precision.HIGHEST,
                               preferred_element_type=jnp.float32)

    def onehot_at(b):
        return (ti_ref[b] == lane_i).astype(jnp.float32)

    counts = cnt_ref[...]
    ci = counts.astype(jnp.int32)
    padded = ((ci + (BLK - 1)) // BLK) * BLK
    padded = jnp.where(lane_i >= NE, 0, padded)
    ptot = jnp.sum(padded, axis=1, keepdims=True)
    padded = padded + jnp.where(lane_i == NE, NPAD - ptot, 0)
    padded_f = padded.astype(jnp.float32)
    offsets = _hi(padded_f, u_strict)
    offsets_col = t_col(offsets)

    def p2(b, carry):
        oh = onehot_at(b)
        cum = jnp.dot(l_strict, oh,
                      preferred_element_type=jnp.float32)
        rank = jnp.sum(oh * (cum + carry), axis=1, keepdims=True)
        dest = rank + jnp.dot(oh, offsets_col,
                              preferred_element_type=jnp.float32)
        dest_ref[b] = dest.astype(jnp.int32)
        return carry + jnp.sum(oh, axis=0, keepdims=True)

    lax.fori_loop(0, n_blocks, p2, jnp.zeros((1, LANES), jnp.float32))

    bs = offsets / float(BLK)
    mje = jnp.logical_and(iota_col >= bs, lane_row < NE).astype(jnp.float32)
    be = _hi(mje, ones_col) - 1.0
    be_ref[...] = jnp.clip(be, 0.0, NE - 1.0).astype(jnp.int32)
    nreal_ref[...] = (ptot // BLK).astype(jnp.int32)

    scale = NE / (float(K) * T * T)
    aux_ref[...] = jnp.sum(pcol_ref[...] * counts,
                           axis=1, keepdims=True) * scale


def _dispatch(ti3d, pcol, cnt):
    return pl.pallas_call(
        _dispatch_body,
        out_shape=(
            jax.ShapeDtypeStruct((TK // RB, RB, 1), jnp.int32),
            jax.ShapeDtypeStruct((LANES, 1), jnp.int32),
            jax.ShapeDtypeStruct((1, 1), jnp.int32),
            jax.ShapeDtypeStruct((1, 1), jnp.float32),
        ),
    )(ti3d, pcol, cnt)


_STRIPE = NPAD // 16
_NCH = 12


def _scatter_body(x_hbm, dest3_hbm, destw_hbm, tww_hbm,
                  xg_hbm, wpart_hbm,
                  didx_v, didxw_v, tww_v, rows_v, rows2_v, wz_v,
                  sem, sem2, semw, spw):
    cid = lax.axis_index("c")
    sid = lax.axis_index("s")
    wid = sid * 2 + cid
    pltpu.sync_copy(dest3_hbm.at[wid], didx_v)
    pltpu.sync_copy(destw_hbm.at[wid], didxw_v)
    pltpu.sync_copy(tww_hbm.at[wid], tww_v)
    for i in range(_STRIPE // 16):
        wz_v[pl.ds(i * 16, 16)] = jnp.zeros((16,), jnp.float32)
    pltpu.sync_copy(wz_v, spw.at[pl.ds(sid * _STRIPE, _STRIPE)])
    plsc.subcore_barrier()
    wcps = [pltpu.async_copy(tww_v.at[j], spw.at[didxw_v.at[j]], semw,
                             add=True)
            for j in range(K)]
    rows = (rows_v, rows2_v)
    sems = (sem, sem2)
    pltpu.sync_copy(x_hbm.at[pl.ds(wid * 128, 32)], rows_v)
    pending = {}
    for c in range(4):
        cur = rows[c % 2]
        pending[c] = [
            pltpu.async_copy(cur, xg_hbm.at[didx_v.at[k * 4 + c]],
                             sems[c % 2])
            for k in range(K)]
        if c + 1 < 4:
            if c - 1 in pending:
                for cp in pending.pop(c - 1):
                    cp.wait()
            pltpu.sync_copy(x_hbm.at[pl.ds(wid * 128 + (c + 1) * 32, 32)],
                            rows[(c + 1) % 2])
    for c in sorted(pending):
        for cp in pending[c]:
            cp.wait()
    for cp in wcps:
        cp.wait()
    plsc.subcore_barrier()
    pltpu.sync_copy(spw.at[pl.ds(sid * _STRIPE, _STRIPE)], wz_v)
    off = pl.multiple_of(cid * NPAD + sid * _STRIPE, 8)
    pltpu.sync_copy(wz_v, wpart_hbm.at[pl.ds(off, _STRIPE)])


def _scatter_rows(x, dest3, destw, tww):
    return pl.kernel(
        _scatter_body,
        out_type=(
            jax.ShapeDtypeStruct((NPAD, D), jnp.float32),
            jax.ShapeDtypeStruct((2 * NPAD,), jnp.float32),
        ),
        mesh=plsc.VectorSubcoreMesh(core_axis_name="c", subcore_axis_name="s"),
        scratch_types=[
            pltpu.VMEM((_NCH, 32), jnp.int32),
            pltpu.VMEM((K, LANES), jnp.int32),
            pltpu.VMEM((K, LANES), jnp.float32),
            pltpu.VMEM((32, D), jnp.float32),
            pltpu.VMEM((32, D), jnp.float32),
            pltpu.VMEM((_STRIPE,), jnp.float32),
            pltpu.SemaphoreType.DMA,
            pltpu.SemaphoreType.DMA,
            pltpu.SemaphoreType.DMA,
            pltpu.VMEM_SHARED((NPAD,), jnp.float32),
        ],
    )(x, dest3, destw, tww)


def _expert_body(be_ref, nb_ref, xg_ref, w1_ref, b1_ref, w2_ref, b2_ref,
                 wv_ref, out_ref):
    @pl.when(pl.program_id(0) < nb_ref[0])
    def _():
        x = xg_ref[...]
        h = jnp.dot(x, w1_ref[0], preferred_element_type=jnp.float32)
        h = _gelu(h + b1_ref[0])
        o = jnp.dot(h, w2_ref[0], preferred_element_type=jnp.float32)
        o = o + b2_ref[0]
        out_ref[...] = o * (wv_ref[0, 0] + wv_ref[1, 0])


def _expert_mlp(be, nb, xg, wu1, bu1, wu2, bu2, wpart4d):
    grid_spec = pltpu.PrefetchScalarGridSpec(
        num_scalar_prefetch=2,
        grid=(NB,),
        in_specs=[
            pl.BlockSpec((BLK, D), lambda i, be, nb: (i, 0)),
            pl.BlockSpec((1, D, H), lambda i, be, nb: (be[i], 0, 0)),
            pl.BlockSpec((1, 1, H), lambda i, be, nb: (be[i], 0, 0)),
            pl.BlockSpec((1, H, D), lambda i, be, nb: (be[i], 0, 0)),
            pl.BlockSpec((1, 1, D), lambda i, be, nb: (be[i], 0, 0)),
            pl.BlockSpec((2, 1, BLK, 1), lambda i, be, nb: (0, i, 0, 0)),
        ],
        out_specs=pl.BlockSpec((BLK, D), lambda i, be, nb: (i, 0)),
    )
    return pl.pallas_call(
        _expert_body,
        grid_spec=grid_spec,
        out_shape=jax.ShapeDtypeStruct((NPAD, D), jnp.float32),
        compiler_params=pltpu.CompilerParams(
            dimension_semantics=("arbitrary",)),
    )(be, nb, xg, wu1, bu1.reshape(NE, 1, H), wu2, bu2.reshape(NE, 1, D),
      wpart4d)


def _shared_body(x_ref, w1_ref, b1_ref, w2_ref, b2_ref, out_ref):
    h = jnp.dot(x_ref[...], w1_ref[...], preferred_element_type=jnp.float32)
    h = _gelu(h + b1_ref[...])
    o = jnp.dot(h, w2_ref[...], preferred_element_type=jnp.float32)
    out_ref[...] = o + b2_ref[...]


def _shared_mlp(x, ws1, bs1, ws2, bs2):
    return pl.pallas_call(
        _shared_body,
        grid=(T // TBLK,),
        in_specs=[
            pl.BlockSpec((TBLK, D), lambda i: (i, 0)),
            pl.BlockSpec((D, H), lambda i: (0, 0)),
            pl.BlockSpec((1, H), lambda i: (0, 0)),
            pl.BlockSpec((H, D), lambda i: (0, 0)),
            pl.BlockSpec((1, D), lambda i: (0, 0)),
        ],
        out_specs=pl.BlockSpec((TBLK, D), lambda i: (i, 0)),
        out_shape=jax.ShapeDtypeStruct((T, D), jnp.float32),
    )(x, ws1, bs1.reshape(1, H), ws2, bs2.reshape(1, D))


_CTOK = 16


def _combine_body(dest_hbm, oute_hbm, shared_hbm, res_hbm,
                  didx_v, g0, g1, acc_v, sem0, sem1, wsem):
    wid = lax.axis_index("s") * 2 + lax.axis_index("c")
    per_tok = T // NW
    n_chunks = per_tok // _CTOK
    pltpu.sync_copy(dest_hbm.at[pl.ds(wid * per_tok * K, per_tok * K)],
                    didx_v)
    bufs = (g0, g1)
    sems = (sem0, sem1)

    def start(c):
        return pltpu.async_copy(
            oute_hbm.at[didx_v.at[pl.ds(c * _CTOK * K, _CTOK * K)]],
            bufs[c % 2], sems[c % 2])

    cps = {0: start(0)}
    wcp = None
    for c in range(n_chunks):
        if c + 1 < n_chunks:
            cps[c + 1] = start(c + 1)
        tbase = wid * per_tok + c * _CTOK
        if wcp is not None:
            wcp.wait()
        pltpu.sync_copy(shared_hbm.at[pl.ds(tbase, _CTOK)], acc_v)
        cps[c].wait()
        gath_v = bufs[c % 2]

        @plsc.parallel_loop(0, _CTOK, unroll=2)
        def add_tok(u):
            for j in range(D // 16):
                s = pl.ds(j * 16, 16)
                acc_v[u, s] = (acc_v[u, s] + gath_v[u, s]
                               + gath_v[_CTOK + u, s]
                               + gath_v[2 * _CTOK + u, s])

        wcp = pltpu.async_copy(acc_v, res_hbm.at[pl.ds(tbase, _CTOK)], wsem)
    wcp.wait()


def _combine(dest_flat, oute, shared):
    return pl.kernel(
        _combine_body,
        out_type=jax.ShapeDtypeStruct((T, D), jnp.float32),
        mesh=plsc.VectorSubcoreMesh(core_axis_name="c", subcore_axis_name="s"),
        scratch_types=[
            pltpu.VMEM((T // NW * K,), jnp.int32),
            pltpu.VMEM((_CTOK * K, D), jnp.float32),
            pltpu.VMEM((_CTOK * K, D), jnp.float32),
            pltpu.VMEM((_CTOK, D), jnp.float32),
            pltpu.SemaphoreType.DMA,
            pltpu.SemaphoreType.DMA,
            pltpu.SemaphoreType.DMA,
        ],
    )(dest_flat, oute, shared)


@jax.jit
def kernel(x, Wg, bg, Wu1, bu1, Wu2, bu2, Ws1, bs1, Ws2, bs2, bias):
    o_shape = x.shape
    xf = x.reshape(-1, o_shape[-1])

    wg_p = jnp.zeros((D, LANES), jnp.float32).at[:, :NE].set(Wg)
    bg_p = jnp.zeros((1, LANES), jnp.float32).at[0, :NE].set(bg)
    bias_p = jnp.full((1, LANES), _NEG, jnp.float32).at[0, :NE].set(bias)

    i0, i1, i2, w0, w1, w2, pcol, cnt = _gate_topk(xf, wg_p, bg_p, bias_p)

    ti = jnp.concatenate([i0, i1, i2], axis=1).reshape(-1)
    tw = jnp.concatenate([w0, w1, w2], axis=1).reshape(-1)

    dest, be, nreal, aux = _dispatch(ti.reshape(TK // RB, RB, 1),
                                     pcol, cnt)
    dest_flat = dest.reshape(-1)

    dest3 = dest_flat.reshape(T, K).T.reshape(K, NW, 4, 32) \
        .transpose(1, 0, 2, 3).reshape(NW, _NCH, 32)
    destw = dest_flat.reshape(NW, K, LANES)
    tww = tw.reshape(NW, K, LANES)

    shared = _shared_mlp(xf, Ws1, bs1, Ws2, bs2)
    xg, wpart = _scatter_rows(xf, dest3, destw, tww)

    n_ch = (T // NW) // _CTOK
    dest_g = dest_flat.reshape(NW, n_ch, _CTOK, K).transpose(0, 1, 3, 2) \
        .reshape(-1)

    oute = _expert_mlp(be.reshape(-1)[:NB], nreal.reshape(-1), xg,
                       Wu1, bu1, Wu2, bu2, wpart.reshape(2, NB, BLK, 1))
    result = _combine(dest_g, oute, shared)

    return result.reshape(o_shape), aux[0, 0]
